# Initial kernel scaffold; baseline (speedup 1.0000x reference)
import jax
import jax.numpy as jnp
from jax.experimental import pallas as pl


def kernel(hidden_states, gate_w, w1, w2, w3):
    raise NotImplementedError("write your pallas kernel here")


# speedup vs baseline: 1.0379x; 1.0379x over previous
import functools

import jax
import jax.numpy as jnp
from jax import lax
from jax.experimental import pallas as pl
from jax.experimental.pallas import tpu as pltpu
from jax.experimental.pallas import tpu_sc as plsc

H = 1024
F = 3584
E = 8
T = 2048
K = 2
NSLOT = T * K
BM = 128
NB = NSLOT // BM + E
S = NB * BM
EL = 128
NW = 32
CH = NSLOT // NW
TPW = T // NW
BF = 896
NFT = F // BF
BE_PAD = 48


def _router_body(x_ref, gw_ref, seg_ref, w_ref, e_ref, hist_ref):
    x = x_ref[...].astype(jnp.bfloat16)
    gw = gw_ref[...].astype(jnp.bfloat16)
    logits = lax.dot_general(x, gw, (((1,), (1,)), ((), ())),
                             preferred_element_type=jnp.float32)
    lane = lax.broadcasted_iota(jnp.int32, (T, EL), 1)
    valid = lane < E
    logits = jnp.where(valid, logits, -1e30)
    m = jnp.max(logits, axis=1, keepdims=True)
    p = jnp.exp(logits - m)
    p = jnp.where(valid, p, 0.0)
    probs = p / jnp.sum(p, axis=1, keepdims=True)
    a1 = jnp.max(probs, axis=1, keepdims=True)
    e1 = jnp.min(jnp.where(probs >= a1, lane, EL), axis=1, keepdims=True)
    probs2 = jnp.where(lane == e1, -1.0, probs)
    a2 = jnp.max(probs2, axis=1, keepdims=True)
    e2 = jnp.min(jnp.where(probs2 >= a2, lane, EL), axis=1, keepdims=True)
    tot = a1 + a2
    w_ref[...] = jnp.where(lane == 0, a1 / tot,
                           jnp.where(lane == 1, a2 / tot, 0.0))
    e_ref[...] = jnp.where(lane == 0, e1, jnp.where(lane == 1, e2, 0))
    oh = ((e1 == lane).astype(jnp.float32) + (e2 == lane).astype(jnp.float32))
    oh = jnp.where(valid, oh, 0.0)
    seg = seg_ref[...]
    hist_ref[...] = lax.dot_general(seg, oh, (((1,), (0,)), ((), ())),
                                    preferred_element_type=jnp.float32)


def _router(x, gw_pad, seg):
    return pl.pallas_call(
        _router_body,
        out_shape=(
            jax.ShapeDtypeStruct((T, EL), jnp.float32),
            jax.ShapeDtypeStruct((T, EL), jnp.int32),
            jax.ShapeDtypeStruct((NW, EL), jnp.float32),
        ),
    )(x, gw_pad, seg)


def _dispatch_body(ids_hbm, hist_hbm, x_hbm, pos_hbm, be_hbm, xs_hbm,
                   ids_v, hist_v, pos_v, base_ref, idx_e, idx_o,
                   rows_v, be_v, sem):
    cid = lax.axis_index("c")
    sid = lax.axis_index("s")
    wid = sid * 2 + cid
    base_slot = wid * CH
    pltpu.sync_copy(ids_hbm.at[pl.ds(base_slot, CH)], ids_v)
    pltpu.sync_copy(hist_hbm, hist_v)
    lanei = lax.iota(jnp.int32, 16)

    def _tot_step(w, acc):
        return acc + hist_v[pl.ds(w * 16, 16)]
    nvec = lax.fori_loop(0, NW, _tot_step, jnp.zeros((16,), jnp.int32))
    pb = (nvec + (BM - 1)) >> 7
    csum = plsc.cumsum(pb)
    bstart = csum - pb
    segstart = bstart << 7
    prefix = lax.fori_loop(0, wid, _tot_step, jnp.zeros((16,), jnp.int32))
    base_vec = segstart + prefix

    for i in range(CH // 16):
        v = ids_v[pl.ds(i * 16, 16)]
        base_ref[...] = base_vec
        bgat = plsc.load_gather(base_ref, [v])
        rank = jnp.zeros((16,), jnp.int32)
        hv = jnp.zeros((16,), jnp.int32)
        for e in range(E):
            m = v == e
            mi = jnp.where(m, 1, 0)
            rank = jnp.where(m, plsc.cumsum(mi), rank)
            hv = jnp.where(lanei == e, jnp.sum(mi), hv)
        pos_v[pl.ds(i * 16, 16)] = bgat + rank - 1
        base_vec = base_vec + hv
    pltpu.sync_copy(pos_v, pos_hbm.at[pl.ds(base_slot, CH)])

    for i in range(TPW // 16):
        j2 = (lax.iota(jnp.int32, 16) + i * 16) * 2
        idx_e[pl.ds(i * 16, 16)] = plsc.load_gather(pos_v, [j2])
        idx_o[pl.ds(i * 16, 16)] = plsc.load_gather(pos_v, [j2 + 1])
    pltpu.sync_copy(x_hbm.at[pl.ds(wid * TPW, TPW)], rows_v)
    pltpu.async_copy(rows_v, xs_hbm.at[idx_e], sem).wait()
    pltpu.async_copy(rows_v, xs_hbm.at[idx_o], sem).wait()

    @pl.when(wid == 0)
    def _():
        for i in range(BE_PAD // 16):
            bidx = lax.iota(jnp.int32, 16) + i * 16
            acc = jnp.zeros((16,), jnp.int32)
            for e in range(E):
                se = jnp.sum(jnp.where(lanei == e, bstart, 0))
                acc = acc + jnp.where(bidx >= se, 1, 0)
            be_v[pl.ds(i * 16, 16)] = jnp.maximum(acc - 1, 0)
        pltpu.sync_copy(be_v, be_hbm)


def _dispatch(ids, hist, x):
    mesh = plsc.VectorSubcoreMesh(core_axis_name="c", subcore_axis_name="s")
    fn = pl.kernel(
        _dispatch_body,
        out_type=(
            jax.ShapeDtypeStruct((NSLOT,), jnp.int32),
            jax.ShapeDtypeStruct((BE_PAD,), jnp.int32),
            jax.ShapeDtypeStruct((S, H), jnp.float32),
        ),
        mesh=mesh,
        scratch_types=(
            pltpu.VMEM((CH,), jnp.int32),
            pltpu.VMEM((NW * 16,), jnp.int32),
            pltpu.VMEM((CH,), jnp.int32),
            pltpu.VMEM((16,), jnp.int32),
            pltpu.VMEM((TPW,), jnp.int32),
            pltpu.VMEM((TPW,), jnp.int32),
            pltpu.VMEM((TPW, H), jnp.float32),
            pltpu.VMEM((BE_PAD,), jnp.int32),
            pltpu.SemaphoreType.DMA,
        ),
        compiler_params=pltpu.CompilerParams(needs_layout_passes=False),
    )
    return fn(ids, hist, x)


def _ffn_body(be_ref, xs_ref, w1_ref, w3_ref, w2_ref, out_ref):
    j = pl.program_id(1)
    x = xs_ref[...]
    w1b = w1_ref[0]
    w3b = w3_ref[0]
    w2b = w2_ref[0]
    x16 = x.astype(jnp.bfloat16)
    t1 = lax.dot_general(x16, w1b.astype(jnp.bfloat16),
                         (((1,), (1,)), ((), ())),
                         preferred_element_type=jnp.float32)
    t3 = lax.dot_general(x16, w3b.astype(jnp.bfloat16),
                         (((1,), (1,)), ((), ())),
                         preferred_element_type=jnp.float32)
    h = (t1 / (1.0 + jnp.exp(-t1))) * t3
    o = lax.dot_general(h.astype(jnp.bfloat16), w2b.astype(jnp.bfloat16),
                        (((1,), (1,)), ((), ())),
                        preferred_element_type=jnp.float32)

    @pl.when(j == 0)
    def _():
        out_ref[...] = o

    @pl.when(j != 0)
    def _():
        out_ref[...] += o


def _ffn(be, xs, w1, w3, w2):
    grid_spec = pltpu.PrefetchScalarGridSpec(
        num_scalar_prefetch=1,
        grid=(NB, NFT),
        in_specs=[
            pl.BlockSpec((BM, H), lambda i, j, be: (i, 0)),
            pl.BlockSpec((1, BF, H), lambda i, j, be: (be[i], j, 0)),
            pl.BlockSpec((1, BF, H), lambda i, j, be: (be[i], j, 0)),
            pl.BlockSpec((1, H, BF), lambda i, j, be: (be[i], 0, j)),
        ],
        out_specs=pl.BlockSpec((BM, H), lambda i, j, be: (i, 0)),
    )
    return pl.pallas_call(
        _ffn_body,
        grid_spec=grid_spec,
        out_shape=jax.ShapeDtypeStruct((S, H), jnp.float32),
        compiler_params=pltpu.CompilerParams(
            dimension_semantics=("arbitrary", "arbitrary")),
    )(be, xs, w1, w3, w2)


def _combine_body(ys_hbm, pos_hbm, w_hbm, out_hbm,
                  pos_v, w_v, idx_v, rows_v, out_v, sem):
    cid = lax.axis_index("c")
    sid = lax.axis_index("s")
    wid = sid * 2 + cid
    base_slot = wid * CH
    base_tok = wid * TPW
    pltpu.sync_copy(pos_hbm.at[pl.ds(base_slot, CH)], pos_v)
    pltpu.sync_copy(w_hbm.at[pl.ds(base_slot, CH)], w_v)
    for g in range(4):
        for i in range(2):
            idx_v[pl.ds(i * 16, 16)] = pos_v[pl.ds(g * 32 + i * 16, 16)]
        pltpu.async_copy(ys_hbm.at[idx_v], rows_v, sem).wait()
        lanei = lax.iota(jnp.int32, 16)
        for r in range(16):
            s0 = g * 32 + 2 * r
            wchunk = w_v[pl.ds((s0 // 16) * 16, 16)]
            w0 = jnp.sum(jnp.where(lanei == s0 % 16, wchunk, 0.0))
            w1_ = jnp.sum(jnp.where(lanei == s0 % 16 + 1, wchunk, 0.0))

            def _col(c, _):
                a = rows_v[2 * r, pl.ds(c * 16, 16)]
                b = rows_v[2 * r + 1, pl.ds(c * 16, 16)]
                out_v[r, pl.ds(c * 16, 16)] = w0 * a + w1_ * b
                return 0
            lax.fori_loop(0, H // 16, _col, 0)
        pltpu.sync_copy(out_v, out_hbm.at[pl.ds(base_tok + g * 16, 16)])


def _combine(ys, pos, w):
    mesh = plsc.VectorSubcoreMesh(core_axis_name="c", subcore_axis_name="s")
    fn = pl.kernel(
        _combine_body,
        out_type=jax.ShapeDtypeStruct((T, H), jnp.float32),
        mesh=mesh,
        scratch_types=(
            pltpu.VMEM((CH,), jnp.int32),
            pltpu.VMEM((CH,), jnp.float32),
            pltpu.VMEM((32,), jnp.int32),
            pltpu.VMEM((32, H), jnp.float32),
            pltpu.VMEM((16, H), jnp.float32),
            pltpu.SemaphoreType.DMA,
        ),
        compiler_params=pltpu.CompilerParams(needs_layout_passes=False),
    )
    return fn(ys, pos, w)


@jax.jit
def _run(hidden_states, gate_w, w1, w2, w3):
    x = hidden_states.reshape(T, H)
    gw_pad = jnp.zeros((EL, H), jnp.float32).at[:E].set(gate_w)
    tok = lax.broadcasted_iota(jnp.int32, (NW, T), 1)
    chunk = lax.broadcasted_iota(jnp.int32, (NW, T), 0)
    seg = (tok // TPW == chunk).astype(jnp.float32)
    wout, eout, hist = _router(x, gw_pad, seg)
    topw = wout[:, :K].reshape(-1)
    ids = eout[:, :K].reshape(-1)
    hist_i = hist[:, :16].astype(jnp.int32).reshape(-1)
    pos, be, xs = _dispatch(ids, hist_i, x)
    ys = _ffn(be, xs, w1, w3, w2)
    final = _combine(ys, pos, topw)
    return final.reshape(1, T, H)


def kernel(hidden_states, gate_w, w1, w2, w3):
    return _run(hidden_states, gate_w, w1, w2, w3)


# speedup vs baseline: 1.1355x; 1.0940x over previous
import functools

import jax
import jax.numpy as jnp
from jax import lax
from jax.experimental import pallas as pl
from jax.experimental.pallas import tpu as pltpu
from jax.experimental.pallas import tpu_sc as plsc

H = 1024
F = 3584
E = 8
T = 2048
K = 2
NSLOT = T * K
BM = 128
NB = NSLOT // BM + E
S = NB * BM
EL = 128
NW = 32
CH = NSLOT // NW
TPW = T // NW
BF = 896
NFT = F // BF
BE_PAD = 48


def _router_body(x_ref, gw_ref, seg_ref, w_ref, e_ref, hist_ref):
    x = x_ref[...].astype(jnp.bfloat16)
    gw = gw_ref[...].astype(jnp.bfloat16)
    logits = lax.dot_general(x, gw, (((1,), (1,)), ((), ())),
                             preferred_element_type=jnp.float32)
    lane = lax.broadcasted_iota(jnp.int32, (T, EL), 1)
    valid = lane < E
    logits = jnp.where(valid, logits, -1e30)
    m = jnp.max(logits, axis=1, keepdims=True)
    p = jnp.exp(logits - m)
    p = jnp.where(valid, p, 0.0)
    probs = p / jnp.sum(p, axis=1, keepdims=True)
    a1 = jnp.max(probs, axis=1, keepdims=True)
    e1 = jnp.min(jnp.where(probs >= a1, lane, EL), axis=1, keepdims=True)
    probs2 = jnp.where(lane == e1, -1.0, probs)
    a2 = jnp.max(probs2, axis=1, keepdims=True)
    e2 = jnp.min(jnp.where(probs2 >= a2, lane, EL), axis=1, keepdims=True)
    tot = a1 + a2
    w_ref[...] = jnp.where(lane == 0, a1 / tot,
                           jnp.where(lane == 1, a2 / tot, 0.0))
    e_ref[...] = jnp.where(lane == 0, e1, jnp.where(lane == 1, e2, 0))
    oh = ((e1 == lane).astype(jnp.float32) + (e2 == lane).astype(jnp.float32))
    oh = jnp.where(valid, oh, 0.0)
    seg = seg_ref[...]
    hist_ref[...] = lax.dot_general(seg, oh, (((1,), (0,)), ((), ())),
                                    preferred_element_type=jnp.float32)


def _router(x, gw_pad, seg):
    return pl.pallas_call(
        _router_body,
        out_shape=(
            jax.ShapeDtypeStruct((T, EL), jnp.float32),
            jax.ShapeDtypeStruct((T, EL), jnp.int32),
            jax.ShapeDtypeStruct((NW, EL), jnp.float32),
        ),
    )(x, gw_pad, seg)


def _dispatch_body(ids_hbm, hist_hbm, x_hbm, pos_hbm, be_hbm, xs_hbm,
                   ids_v, hist_v, pos_v, base_ref, idx_e, idx_o,
                   rows_v, be_v, sem):
    cid = lax.axis_index("c")
    sid = lax.axis_index("s")
    wid = sid * 2 + cid
    base_slot = wid * CH
    pltpu.sync_copy(ids_hbm.at[pl.ds(base_slot, CH)], ids_v)
    pltpu.sync_copy(hist_hbm, hist_v)
    lanei = lax.iota(jnp.int32, 16)

    def _tot_step(w, acc):
        return acc + hist_v[pl.ds(w * 16, 16)]
    nvec = lax.fori_loop(0, NW, _tot_step, jnp.zeros((16,), jnp.int32))
    pb = (nvec + (BM - 1)) >> 7
    csum = plsc.cumsum(pb)
    bstart = csum - pb
    segstart = bstart << 7
    prefix = lax.fori_loop(0, wid, _tot_step, jnp.zeros((16,), jnp.int32))
    base_vec = segstart + prefix

    for i in range(CH // 16):
        v = ids_v[pl.ds(i * 16, 16)]
        base_ref[...] = base_vec
        bgat = plsc.load_gather(base_ref, [v])
        rank = jnp.zeros((16,), jnp.int32)
        hv = jnp.zeros((16,), jnp.int32)
        for e in range(E):
            m = v == e
            mi = jnp.where(m, 1, 0)
            rank = jnp.where(m, plsc.cumsum(mi), rank)
            hv = jnp.where(lanei == e, jnp.sum(mi), hv)
        pos_v[pl.ds(i * 16, 16)] = bgat + rank - 1
        base_vec = base_vec + hv
    pltpu.sync_copy(pos_v, pos_hbm.at[pl.ds(base_slot, CH)])

    for i in range(TPW // 16):
        j2 = (lax.iota(jnp.int32, 16) + i * 16) * 2
        idx_e[pl.ds(i * 16, 16)] = plsc.load_gather(pos_v, [j2])
        idx_o[pl.ds(i * 16, 16)] = plsc.load_gather(pos_v, [j2 + 1])
    pltpu.sync_copy(x_hbm.at[pl.ds(wid * TPW, TPW)], rows_v)
    pltpu.async_copy(rows_v, xs_hbm.at[idx_e], sem).wait()
    pltpu.async_copy(rows_v, xs_hbm.at[idx_o], sem).wait()

    @pl.when(wid == 0)
    def _():
        for i in range(BE_PAD // 16):
            bidx = lax.iota(jnp.int32, 16) + i * 16
            acc = jnp.zeros((16,), jnp.int32)
            for e in range(E):
                se = jnp.sum(jnp.where(lanei == e, bstart, 0))
                acc = acc + jnp.where(bidx >= se, 1, 0)
            be_v[pl.ds(i * 16, 16)] = jnp.maximum(acc - 1, 0)
        pltpu.sync_copy(be_v, be_hbm)


def _dispatch(ids, hist, x):
    mesh = plsc.VectorSubcoreMesh(core_axis_name="c", subcore_axis_name="s")
    fn = pl.kernel(
        _dispatch_body,
        out_type=(
            jax.ShapeDtypeStruct((NSLOT,), jnp.int32),
            jax.ShapeDtypeStruct((BE_PAD,), jnp.int32),
            jax.ShapeDtypeStruct((S, H), jnp.float32),
        ),
        mesh=mesh,
        scratch_types=(
            pltpu.VMEM((CH,), jnp.int32),
            pltpu.VMEM((NW * 16,), jnp.int32),
            pltpu.VMEM((CH,), jnp.int32),
            pltpu.VMEM((16,), jnp.int32),
            pltpu.VMEM((TPW,), jnp.int32),
            pltpu.VMEM((TPW,), jnp.int32),
            pltpu.VMEM((TPW, H), jnp.float32),
            pltpu.VMEM((BE_PAD,), jnp.int32),
            pltpu.SemaphoreType.DMA,
        ),
        compiler_params=pltpu.CompilerParams(needs_layout_passes=False),
    )
    return fn(ids, hist, x)


def _ffn_body(be_ref, xs_ref, w1_ref, w3_ref, w2_ref, out_ref):
    x16 = xs_ref[...].astype(jnp.bfloat16)
    w1b = w1_ref[0]
    w3b = w3_ref[0]
    w2b = w2_ref[0]
    t1 = lax.dot_general(x16, w1b, (((1,), (1,)), ((), ())),
                         preferred_element_type=jnp.float32)
    t3 = lax.dot_general(x16, w3b, (((1,), (1,)), ((), ())),
                         preferred_element_type=jnp.float32)
    h = (t1 / (1.0 + jnp.exp(-t1))) * t3
    out_ref[...] = lax.dot_general(
        h.astype(jnp.bfloat16), w2b, (((1,), (1,)), ((), ())),
        preferred_element_type=jnp.float32)


def _ffn(be, xs, w1, w3, w2):
    grid_spec = pltpu.PrefetchScalarGridSpec(
        num_scalar_prefetch=1,
        grid=(NB,),
        in_specs=[
            pl.BlockSpec((BM, H), lambda i, be: (i, 0)),
            pl.BlockSpec((1, F, H), lambda i, be: (be[i], 0, 0)),
            pl.BlockSpec((1, F, H), lambda i, be: (be[i], 0, 0)),
            pl.BlockSpec((1, H, F), lambda i, be: (be[i], 0, 0)),
        ],
        out_specs=pl.BlockSpec((BM, H), lambda i, be: (i, 0)),
    )
    return pl.pallas_call(
        _ffn_body,
        grid_spec=grid_spec,
        out_shape=jax.ShapeDtypeStruct((S, H), jnp.float32),
        compiler_params=pltpu.CompilerParams(
            dimension_semantics=("arbitrary",)),
    )(be, xs, w1, w3, w2)


def _combine_body(ys_hbm, pos_hbm, w_hbm, out_hbm,
                  pos_v, w_v, idx_v, rows_v, out_v, sem):
    cid = lax.axis_index("c")
    sid = lax.axis_index("s")
    wid = sid * 2 + cid
    base_slot = wid * CH
    base_tok = wid * TPW
    pltpu.sync_copy(pos_hbm.at[pl.ds(base_slot, CH)], pos_v)
    pltpu.sync_copy(w_hbm.at[pl.ds(base_slot, CH)], w_v)
    for g in range(4):
        for i in range(2):
            idx_v[pl.ds(i * 16, 16)] = pos_v[pl.ds(g * 32 + i * 16, 16)]
        pltpu.async_copy(ys_hbm.at[idx_v], rows_v, sem).wait()
        lanei = lax.iota(jnp.int32, 16)
        for r in range(16):
            s0 = g * 32 + 2 * r
            wchunk = w_v[pl.ds((s0 // 16) * 16, 16)]
            w0 = jnp.sum(jnp.where(lanei == s0 % 16, wchunk, 0.0))
            w1_ = jnp.sum(jnp.where(lanei == s0 % 16 + 1, wchunk, 0.0))

            def _col(c, _):
                a = rows_v[2 * r, pl.ds(c * 16, 16)]
                b = rows_v[2 * r + 1, pl.ds(c * 16, 16)]
                out_v[r, pl.ds(c * 16, 16)] = w0 * a + w1_ * b
                return 0
            lax.fori_loop(0, H // 16, _col, 0)
        pltpu.sync_copy(out_v, out_hbm.at[pl.ds(base_tok + g * 16, 16)])


def _combine(ys, pos, w):
    mesh = plsc.VectorSubcoreMesh(core_axis_name="c", subcore_axis_name="s")
    fn = pl.kernel(
        _combine_body,
        out_type=jax.ShapeDtypeStruct((T, H), jnp.float32),
        mesh=mesh,
        scratch_types=(
            pltpu.VMEM((CH,), jnp.int32),
            pltpu.VMEM((CH,), jnp.float32),
            pltpu.VMEM((32,), jnp.int32),
            pltpu.VMEM((32, H), jnp.float32),
            pltpu.VMEM((16, H), jnp.float32),
            pltpu.SemaphoreType.DMA,
        ),
        compiler_params=pltpu.CompilerParams(needs_layout_passes=False),
    )
    return fn(ys, pos, w)


@jax.jit
def _run(hidden_states, gate_w, w1, w2, w3):
    x = hidden_states.reshape(T, H)
    gw_pad = jnp.zeros((EL, H), jnp.float32).at[:E].set(gate_w)
    tok = lax.broadcasted_iota(jnp.int32, (NW, T), 1)
    chunk = lax.broadcasted_iota(jnp.int32, (NW, T), 0)
    seg = (tok // TPW == chunk).astype(jnp.float32)
    wout, eout, hist = _router(x, gw_pad, seg)
    topw = wout[:, :K].reshape(-1)
    ids = eout[:, :K].reshape(-1)
    hist_i = hist[:, :16].astype(jnp.int32).reshape(-1)
    pos, be, xs = _dispatch(ids, hist_i, x)
    ys = _ffn(be, xs, w1.astype(jnp.bfloat16), w3.astype(jnp.bfloat16),
              w2.astype(jnp.bfloat16))
    final = _combine(ys, pos, topw)
    return final.reshape(1, T, H)


def kernel(hidden_states, gate_w, w1, w2, w3):
    return _run(hidden_states, gate_w, w1, w2, w3)


# speedup vs baseline: 1.3386x; 1.1789x over previous
import functools

import jax
import jax.numpy as jnp
from jax import lax
from jax.experimental import pallas as pl
from jax.experimental.pallas import tpu as pltpu
from jax.experimental.pallas import tpu_sc as plsc

H = 1024
F = 3584
E = 8
T = 2048
K = 2
NSLOT = T * K
BM = 128
NB = NSLOT // BM + E
S = NB * BM
EL = 128
NW = 32
CH = NSLOT // NW
TPW = T // NW
F2 = F // 2
BE_PAD = 48


def _router_body(x_ref, gw_ref, seg_ref, w_ref, e_ref, hist_ref):
    x = x_ref[...].astype(jnp.bfloat16)
    gw = gw_ref[...].astype(jnp.bfloat16)
    logits = lax.dot_general(x, gw, (((1,), (1,)), ((), ())),
                             preferred_element_type=jnp.float32)
    lane = lax.broadcasted_iota(jnp.int32, (T, EL), 1)
    valid = lane < E
    logits = jnp.where(valid, logits, -1e30)
    m = jnp.max(logits, axis=1, keepdims=True)
    p = jnp.exp(logits - m)
    p = jnp.where(valid, p, 0.0)
    probs = p / jnp.sum(p, axis=1, keepdims=True)
    a1 = jnp.max(probs, axis=1, keepdims=True)
    e1 = jnp.min(jnp.where(probs >= a1, lane, EL), axis=1, keepdims=True)
    probs2 = jnp.where(lane == e1, -1.0, probs)
    a2 = jnp.max(probs2, axis=1, keepdims=True)
    e2 = jnp.min(jnp.where(probs2 >= a2, lane, EL), axis=1, keepdims=True)
    tot = a1 + a2
    w_ref[...] = jnp.where(lane == 0, a1 / tot,
                           jnp.where(lane == 1, a2 / tot, 0.0))
    e_ref[...] = jnp.where(lane == 0, e1, jnp.where(lane == 1, e2, 0))
    oh = ((e1 == lane).astype(jnp.float32) + (e2 == lane).astype(jnp.float32))
    oh = jnp.where(valid, oh, 0.0)
    seg = seg_ref[...]
    hist_ref[...] = lax.dot_general(seg, oh, (((1,), (0,)), ((), ())),
                                    preferred_element_type=jnp.float32)


def _router(x, gw_pad, seg):
    return pl.pallas_call(
        _router_body,
        out_shape=(
            jax.ShapeDtypeStruct((T, EL), jnp.float32),
            jax.ShapeDtypeStruct((T, EL), jnp.int32),
            jax.ShapeDtypeStruct((NW, EL), jnp.float32),
        ),
    )(x, gw_pad, seg)


def _dispatch_body(ids_hbm, hist_hbm, x_hbm, pos_hbm, be_hbm, xs_hbm,
                   ids_v, hist_v, pos_v, base_ref, idx_e, idx_o,
                   rows_v, be_v, sem):
    cid = lax.axis_index("c")
    sid = lax.axis_index("s")
    wid = sid * 2 + cid
    base_slot = wid * CH
    pltpu.sync_copy(ids_hbm.at[pl.ds(base_slot, CH)], ids_v)
    pltpu.sync_copy(hist_hbm, hist_v)
    lanei = lax.iota(jnp.int32, 16)

    def _tot_step(w, acc):
        return acc + hist_v[pl.ds(w * 16, 16)]
    nvec = lax.fori_loop(0, NW, _tot_step, jnp.zeros((16,), jnp.int32))
    pb = (nvec + (BM - 1)) >> 7
    csum = plsc.cumsum(pb)
    bstart = csum - pb
    segstart = bstart << 7
    prefix = lax.fori_loop(0, wid, _tot_step, jnp.zeros((16,), jnp.int32))
    base_vec = segstart + prefix

    for i in range(CH // 16):
        v = ids_v[pl.ds(i * 16, 16)]
        base_ref[...] = base_vec
        bgat = plsc.load_gather(base_ref, [v])
        rank = jnp.zeros((16,), jnp.int32)
        hv = jnp.zeros((16,), jnp.int32)
        for e in range(E):
            m = v == e
            mi = jnp.where(m, 1, 0)
            rank = jnp.where(m, plsc.cumsum(mi), rank)
            hv = jnp.where(lanei == e, jnp.sum(mi), hv)
        pos_v[pl.ds(i * 16, 16)] = bgat + rank - 1
        base_vec = base_vec + hv
    pltpu.sync_copy(pos_v, pos_hbm.at[pl.ds(base_slot, CH)])

    for i in range(TPW // 16):
        j2 = (lax.iota(jnp.int32, 16) + i * 16) * 2
        idx_e[pl.ds(i * 16, 16)] = plsc.load_gather(pos_v, [j2])
        idx_o[pl.ds(i * 16, 16)] = plsc.load_gather(pos_v, [j2 + 1])
    pltpu.sync_copy(x_hbm.at[pl.ds(wid * TPW, TPW)], rows_v)
    pltpu.async_copy(rows_v, xs_hbm.at[idx_e], sem).wait()
    pltpu.async_copy(rows_v, xs_hbm.at[idx_o], sem).wait()

    @pl.when(wid == 0)
    def _():
        for i in range(BE_PAD // 16):
            bidx = lax.iota(jnp.int32, 16) + i * 16
            acc = jnp.zeros((16,), jnp.int32)
            for e in range(E):
                se = jnp.sum(jnp.where(lanei == e, bstart, 0))
                acc = acc + jnp.where(bidx >= se, 1, 0)
            be_v[pl.ds(i * 16, 16)] = jnp.maximum(acc - 1, 0)
        pltpu.sync_copy(be_v, be_hbm)


def _dispatch(ids, hist, x):
    mesh = plsc.VectorSubcoreMesh(core_axis_name="c", subcore_axis_name="s")
    fn = pl.kernel(
        _dispatch_body,
        out_type=(
            jax.ShapeDtypeStruct((NSLOT,), jnp.int32),
            jax.ShapeDtypeStruct((BE_PAD,), jnp.int32),
            jax.ShapeDtypeStruct((S, H), jnp.float32),
        ),
        mesh=mesh,
        scratch_types=(
            pltpu.VMEM((CH,), jnp.int32),
            pltpu.VMEM((NW * 16,), jnp.int32),
            pltpu.VMEM((CH,), jnp.int32),
            pltpu.VMEM((16,), jnp.int32),
            pltpu.VMEM((TPW,), jnp.int32),
            pltpu.VMEM((TPW,), jnp.int32),
            pltpu.VMEM((TPW, H), jnp.float32),
            pltpu.VMEM((BE_PAD,), jnp.int32),
            pltpu.SemaphoreType.DMA,
        ),
        compiler_params=pltpu.CompilerParams(needs_layout_passes=False),
    )
    return fn(ids, hist, x)


def _ffn_body(be_ref, xs_ref, w1_ref, w3_ref, w2_ref, out_ref):
    x = xs_ref[...]
    w1b = w1_ref[0]
    w3b = w3_ref[0]
    w2b = w2_ref[0]
    t1 = lax.dot_general(x, w1b, (((1,), (1,)), ((), ())),
                         preferred_element_type=jnp.float32)
    t3 = lax.dot_general(x, w3b, (((1,), (1,)), ((), ())),
                         preferred_element_type=jnp.float32)
    h = (t1 / (1.0 + jnp.exp(-t1))) * t3
    out_ref[0] = lax.dot_general(
        h, w2b, (((1,), (1,)), ((), ())),
        preferred_element_type=jnp.float32)


def _ffn(be, xs, w1, w3, w2):
    grid_spec = pltpu.PrefetchScalarGridSpec(
        num_scalar_prefetch=1,
        grid=(2, NB),
        in_specs=[
            pl.BlockSpec((BM, H), lambda j, i, be: (i, 0)),
            pl.BlockSpec((1, F2, H), lambda j, i, be: (be[i], j, 0)),
            pl.BlockSpec((1, F2, H), lambda j, i, be: (be[i], j, 0)),
            pl.BlockSpec((1, H, F2), lambda j, i, be: (be[i], 0, j)),
        ],
        out_specs=pl.BlockSpec((1, BM, H), lambda j, i, be: (j, i, 0)),
    )
    return pl.pallas_call(
        _ffn_body,
        grid_spec=grid_spec,
        out_shape=jax.ShapeDtypeStruct((2, S, H), jnp.float32),
        compiler_params=pltpu.CompilerParams(
            dimension_semantics=("arbitrary", "arbitrary"),
            vmem_limit_bytes=60 * 1024 * 1024),
    )(be, xs, w1, w3, w2)


def _combine_body(ys_hbm, pos_hbm, w_hbm, out_hbm,
                  pos_v, w_v, idx_v, idx2_v, rows_v, rows2_v, out_v, sem):
    cid = lax.axis_index("c")
    sid = lax.axis_index("s")
    wid = sid * 2 + cid
    base_slot = wid * CH
    base_tok = wid * TPW
    pltpu.sync_copy(pos_hbm.at[pl.ds(base_slot, CH)], pos_v)
    pltpu.sync_copy(w_hbm.at[pl.ds(base_slot, CH)], w_v)
    for g in range(4):
        for i in range(2):
            pv = pos_v[pl.ds(g * 32 + i * 16, 16)]
            idx_v[pl.ds(i * 16, 16)] = pv
            idx2_v[pl.ds(i * 16, 16)] = pv + S
        cp1 = pltpu.async_copy(ys_hbm.at[idx_v], rows_v, sem)
        cp2 = pltpu.async_copy(ys_hbm.at[idx2_v], rows2_v, sem)
        cp1.wait()
        cp2.wait()
        lanei = lax.iota(jnp.int32, 16)
        for r in range(16):
            s0 = g * 32 + 2 * r
            wchunk = w_v[pl.ds((s0 // 16) * 16, 16)]
            w0 = jnp.sum(jnp.where(lanei == s0 % 16, wchunk, 0.0))
            w1_ = jnp.sum(jnp.where(lanei == s0 % 16 + 1, wchunk, 0.0))

            def _col(c, _):
                a = rows_v[2 * r, pl.ds(c * 16, 16)] + rows2_v[2 * r, pl.ds(c * 16, 16)]
                b = rows_v[2 * r + 1, pl.ds(c * 16, 16)] + rows2_v[2 * r + 1, pl.ds(c * 16, 16)]
                out_v[r, pl.ds(c * 16, 16)] = w0 * a + w1_ * b
                return 0
            lax.fori_loop(0, H // 16, _col, 0)
        pltpu.sync_copy(out_v, out_hbm.at[pl.ds(base_tok + g * 16, 16)])


def _combine(ys, pos, w):
    mesh = plsc.VectorSubcoreMesh(core_axis_name="c", subcore_axis_name="s")
    fn = pl.kernel(
        _combine_body,
        out_type=jax.ShapeDtypeStruct((T, H), jnp.float32),
        mesh=mesh,
        scratch_types=(
            pltpu.VMEM((CH,), jnp.int32),
            pltpu.VMEM((CH,), jnp.float32),
            pltpu.VMEM((32,), jnp.int32),
            pltpu.VMEM((32,), jnp.int32),
            pltpu.VMEM((32, H), jnp.float32),
            pltpu.VMEM((32, H), jnp.float32),
            pltpu.VMEM((16, H), jnp.float32),
            pltpu.SemaphoreType.DMA,
        ),
        compiler_params=pltpu.CompilerParams(needs_layout_passes=False),
    )
    return fn(ys, pos, w)


@jax.jit
def _run(hidden_states, gate_w, w1, w2, w3):
    x = hidden_states.reshape(T, H)
    gw_pad = jnp.zeros((EL, H), jnp.float32).at[:E].set(gate_w)
    tok = lax.broadcasted_iota(jnp.int32, (NW, T), 1)
    chunk = lax.broadcasted_iota(jnp.int32, (NW, T), 0)
    seg = (tok // TPW == chunk).astype(jnp.float32)
    wout, eout, hist = _router(x, gw_pad, seg)
    topw = wout[:, :K].reshape(-1)
    ids = eout[:, :K].reshape(-1)
    hist_i = hist[:, :16].astype(jnp.int32).reshape(-1)
    pos, be, xs = _dispatch(ids, hist_i, x)
    ys = _ffn(be, xs, w1, w3, w2)
    final = _combine(ys.reshape(2 * S, H), pos, topw)
    return final.reshape(1, T, H)


def kernel(hidden_states, gate_w, w1, w2, w3):
    return _run(hidden_states, gate_w, w1, w2, w3)


# speedup vs baseline: 1.9385x; 1.4481x over previous
import functools

import jax
import jax.numpy as jnp
from jax import lax
from jax.experimental import pallas as pl
from jax.experimental.pallas import tpu as pltpu
from jax.experimental.pallas import tpu_sc as plsc

H = 1024
F = 3584
E = 8
T = 2048
K = 2
NSLOT = T * K
BM = 256
BMLOG = 8
NB = NSLOT // BM + E
S = NB * BM
EL = 128
NW = 32
CH = NSLOT // NW
TPW = T // NW
F2 = F // 2
BE_PAD = 32


def _router_body(x_ref, gw_ref, seg_ref, w_ref, e_ref, hist_ref):
    x = x_ref[...].astype(jnp.bfloat16)
    gw = gw_ref[...].astype(jnp.bfloat16)
    logits = lax.dot_general(x, gw, (((1,), (1,)), ((), ())),
                             preferred_element_type=jnp.float32)
    lane = lax.broadcasted_iota(jnp.int32, (T, EL), 1)
    valid = lane < E
    logits = jnp.where(valid, logits, -1e30)
    m = jnp.max(logits, axis=1, keepdims=True)
    p = jnp.exp(logits - m)
    p = jnp.where(valid, p, 0.0)
    probs = p / jnp.sum(p, axis=1, keepdims=True)
    a1 = jnp.max(probs, axis=1, keepdims=True)
    e1 = jnp.min(jnp.where(probs >= a1, lane, EL), axis=1, keepdims=True)
    probs2 = jnp.where(lane == e1, -1.0, probs)
    a2 = jnp.max(probs2, axis=1, keepdims=True)
    e2 = jnp.min(jnp.where(probs2 >= a2, lane, EL), axis=1, keepdims=True)
    tot = a1 + a2
    w_ref[...] = jnp.where(lane == 0, a1 / tot,
                           jnp.where(lane == 1, a2 / tot, 0.0))
    e_ref[...] = jnp.where(lane == 0, e1, jnp.where(lane == 1, e2, 0))
    oh = ((e1 == lane).astype(jnp.float32) + (e2 == lane).astype(jnp.float32))
    oh = jnp.where(valid, oh, 0.0)
    seg = seg_ref[...]
    hist_ref[...] = lax.dot_general(seg, oh, (((1,), (0,)), ((), ())),
                                    preferred_element_type=jnp.float32)


def _router(x, gw_pad, seg):
    return pl.pallas_call(
        _router_body,
        out_shape=(
            jax.ShapeDtypeStruct((T, EL), jnp.float32),
            jax.ShapeDtypeStruct((T, EL), jnp.int32),
            jax.ShapeDtypeStruct((NW, EL), jnp.float32),
        ),
    )(x, gw_pad, seg)


def _dispatch_body(ids_hbm, hist_hbm, x_hbm, pos_hbm, be_hbm, xs_hbm,
                   ids_v, hist_v, pos_v, base_ref, idx_e, idx_o,
                   rows_v, be_v, sem):
    cid = lax.axis_index("c")
    sid = lax.axis_index("s")
    wid = sid * 2 + cid
    base_slot = wid * CH
    pltpu.sync_copy(ids_hbm.at[pl.ds(base_slot, CH)], ids_v)
    pltpu.sync_copy(hist_hbm, hist_v)
    lanei = lax.iota(jnp.int32, 16)

    def _tot_step(w, acc):
        return acc + hist_v[pl.ds(w * 16, 16)]
    nvec = lax.fori_loop(0, NW, _tot_step, jnp.zeros((16,), jnp.int32))
    pb = (nvec + (BM - 1)) >> BMLOG
    csum = plsc.cumsum(pb)
    bstart = csum - pb
    segstart = bstart << BMLOG
    prefix = lax.fori_loop(0, wid, _tot_step, jnp.zeros((16,), jnp.int32))
    base_vec = segstart + prefix

    for i in range(CH // 16):
        v = ids_v[pl.ds(i * 16, 16)]
        base_ref[...] = base_vec
        bgat = plsc.load_gather(base_ref, [v])
        rank = jnp.zeros((16,), jnp.int32)
        hv = jnp.zeros((16,), jnp.int32)
        for e in range(E):
            m = v == e
            mi = jnp.where(m, 1, 0)
            rank = jnp.where(m, plsc.cumsum(mi), rank)
            hv = jnp.where(lanei == e, jnp.sum(mi), hv)
        pos_v[pl.ds(i * 16, 16)] = bgat + rank - 1
        base_vec = base_vec + hv
    pltpu.sync_copy(pos_v, pos_hbm.at[pl.ds(base_slot, CH)])

    for i in range(TPW // 16):
        j2 = (lax.iota(jnp.int32, 16) + i * 16) * 2
        idx_e[pl.ds(i * 16, 16)] = plsc.load_gather(pos_v, [j2])
        idx_o[pl.ds(i * 16, 16)] = plsc.load_gather(pos_v, [j2 + 1])
    pltpu.sync_copy(x_hbm.at[pl.ds(wid * TPW, TPW)], rows_v)
    pltpu.async_copy(rows_v, xs_hbm.at[idx_e], sem).wait()
    pltpu.async_copy(rows_v, xs_hbm.at[idx_o], sem).wait()

    @pl.when(wid == 0)
    def _():
        for i in range(BE_PAD // 16):
            bidx = lax.iota(jnp.int32, 16) + i * 16
            acc = jnp.zeros((16,), jnp.int32)
            for e in range(E):
                se = jnp.sum(jnp.where(lanei == e, bstart, 0))
                acc = acc + jnp.where(bidx >= se, 1, 0)
            be_v[pl.ds(i * 16, 16)] = jnp.maximum(acc - 1, 0)
        pltpu.sync_copy(be_v, be_hbm)


def _dispatch(ids, hist, x):
    mesh = plsc.VectorSubcoreMesh(core_axis_name="c", subcore_axis_name="s")
    fn = pl.kernel(
        _dispatch_body,
        out_type=(
            jax.ShapeDtypeStruct((NSLOT,), jnp.int32),
            jax.ShapeDtypeStruct((BE_PAD,), jnp.int32),
            jax.ShapeDtypeStruct((S, H), jnp.float32),
        ),
        mesh=mesh,
        scratch_types=(
            pltpu.VMEM((CH,), jnp.int32),
            pltpu.VMEM((NW * 16,), jnp.int32),
            pltpu.VMEM((CH,), jnp.int32),
            pltpu.VMEM((16,), jnp.int32),
            pltpu.VMEM((TPW,), jnp.int32),
            pltpu.VMEM((TPW,), jnp.int32),
            pltpu.VMEM((TPW, H), jnp.float32),
            pltpu.VMEM((BE_PAD,), jnp.int32),
            pltpu.SemaphoreType.DMA,
        ),
        compiler_params=pltpu.CompilerParams(needs_layout_passes=False),
    )
    return fn(ids, hist, x)


def _ffn_body(be_ref, xs_ref, w1_ref, w3_ref, w2_ref, out_ref):
    x = xs_ref[...]
    w1b = w1_ref[0]
    w3b = w3_ref[0]
    w2b = w2_ref[0]
    t1 = lax.dot_general(x, w1b, (((1,), (1,)), ((), ())),
                         preferred_element_type=jnp.float32)
    t3 = lax.dot_general(x, w3b, (((1,), (1,)), ((), ())),
                         preferred_element_type=jnp.float32)
    h = (t1 / (1.0 + jnp.exp(-t1))) * t3
    out_ref[0] = lax.dot_general(
        h, w2b, (((1,), (1,)), ((), ())),
        preferred_element_type=jnp.float32)


def _ffn(be, xs, w1, w3, w2):
    grid_spec = pltpu.PrefetchScalarGridSpec(
        num_scalar_prefetch=1,
        grid=(2, NB),
        in_specs=[
            pl.BlockSpec((BM, H), lambda j, i, be: (i, 0)),
            pl.BlockSpec((1, F2, H), lambda j, i, be: (be[i], j, 0)),
            pl.BlockSpec((1, F2, H), lambda j, i, be: (be[i], j, 0)),
            pl.BlockSpec((1, H, F2), lambda j, i, be: (be[i], 0, j)),
        ],
        out_specs=pl.BlockSpec((1, BM, H), lambda j, i, be: (j, i, 0)),
    )
    return pl.pallas_call(
        _ffn_body,
        grid_spec=grid_spec,
        out_shape=jax.ShapeDtypeStruct((2, S, H), jnp.float32),
        compiler_params=pltpu.CompilerParams(
            dimension_semantics=("arbitrary", "arbitrary"),
            vmem_limit_bytes=60 * 1024 * 1024),
    )(be, xs, w1, w3, w2)


def _combine_body(ys_hbm, pos_hbm, w_hbm, out_hbm,
                  pos_v, w_v, idx_v, idx2_v, rows_v, rows2_v, out_v, sem):
    cid = lax.axis_index("c")
    sid = lax.axis_index("s")
    wid = sid * 2 + cid
    base_slot = wid * CH
    base_tok = wid * TPW
    pltpu.sync_copy(pos_hbm.at[pl.ds(base_slot, CH)], pos_v)
    pltpu.sync_copy(w_hbm.at[pl.ds(base_slot, CH)], w_v)
    for g in range(4):
        for i in range(2):
            pv = pos_v[pl.ds(g * 32 + i * 16, 16)]
            idx_v[pl.ds(i * 16, 16)] = pv
            idx2_v[pl.ds(i * 16, 16)] = pv + S
        cp1 = pltpu.async_copy(ys_hbm.at[idx_v], rows_v, sem)
        cp2 = pltpu.async_copy(ys_hbm.at[idx2_v], rows2_v, sem)
        cp1.wait()
        cp2.wait()
        lanei = lax.iota(jnp.int32, 16)
        for r in range(16):
            s0 = g * 32 + 2 * r
            wchunk = w_v[pl.ds((s0 // 16) * 16, 16)]
            w0 = jnp.sum(jnp.where(lanei == s0 % 16, wchunk, 0.0))
            w1_ = jnp.sum(jnp.where(lanei == s0 % 16 + 1, wchunk, 0.0))

            def _col(c, _):
                a = rows_v[2 * r, pl.ds(c * 16, 16)] + rows2_v[2 * r, pl.ds(c * 16, 16)]
                b = rows_v[2 * r + 1, pl.ds(c * 16, 16)] + rows2_v[2 * r + 1, pl.ds(c * 16, 16)]
                out_v[r, pl.ds(c * 16, 16)] = w0 * a + w1_ * b
                return 0
            lax.fori_loop(0, H // 16, _col, 0)
        pltpu.sync_copy(out_v, out_hbm.at[pl.ds(base_tok + g * 16, 16)])


def _combine(ys, pos, w):
    mesh = plsc.VectorSubcoreMesh(core_axis_name="c", subcore_axis_name="s")
    fn = pl.kernel(
        _combine_body,
        out_type=jax.ShapeDtypeStruct((T, H), jnp.float32),
        mesh=mesh,
        scratch_types=(
            pltpu.VMEM((CH,), jnp.int32),
            pltpu.VMEM((CH,), jnp.float32),
            pltpu.VMEM((32,), jnp.int32),
            pltpu.VMEM((32,), jnp.int32),
            pltpu.VMEM((32, H), jnp.float32),
            pltpu.VMEM((32, H), jnp.float32),
            pltpu.VMEM((16, H), jnp.float32),
            pltpu.SemaphoreType.DMA,
        ),
        compiler_params=pltpu.CompilerParams(needs_layout_passes=False),
    )
    return fn(ys, pos, w)


@jax.jit
def _run(hidden_states, gate_w, w1, w2, w3):
    x = hidden_states.reshape(T, H)
    gw_pad = jnp.zeros((EL, H), jnp.float32).at[:E].set(gate_w)
    tok = lax.broadcasted_iota(jnp.int32, (NW, T), 1)
    chunk = lax.broadcasted_iota(jnp.int32, (NW, T), 0)
    seg = (tok // TPW == chunk).astype(jnp.float32)
    wout, eout, hist = _router(x, gw_pad, seg)
    topw = wout[:, :K].reshape(-1)
    ids = eout[:, :K].reshape(-1)
    hist_i = hist[:, :16].astype(jnp.int32).reshape(-1)
    pos, be, xs = _dispatch(ids, hist_i, x)
    ys = _ffn(be, xs, w1, w3, w2)
    final = _combine(ys.reshape(2 * S, H), pos, topw)
    return final.reshape(1, T, H)


def kernel(hidden_states, gate_w, w1, w2, w3):
    return _run(hidden_states, gate_w, w1, w2, w3)


# speedup vs baseline: 1.9852x; 1.0241x over previous
import functools

import jax
import jax.numpy as jnp
from jax import lax
from jax.experimental import pallas as pl
from jax.experimental.pallas import tpu as pltpu
from jax.experimental.pallas import tpu_sc as plsc

H = 1024
F = 3584
E = 8
T = 2048
K = 2
NSLOT = T * K
BM = 256
BMLOG = 8
NB = NSLOT // BM + E
S = NB * BM
EL = 128
NW = 32
CH = NSLOT // NW
TPW = T // NW
F2 = F // 2
BE_PAD = 32


def _router_body(x_ref, gw_ref, seg_ref, w_ref, e_ref, hist_ref):
    x = x_ref[...].astype(jnp.bfloat16)
    gw = gw_ref[...].astype(jnp.bfloat16)
    logits = lax.dot_general(x, gw, (((1,), (1,)), ((), ())),
                             preferred_element_type=jnp.float32)
    lane = lax.broadcasted_iota(jnp.int32, (T, EL), 1)
    valid = lane < E
    logits = jnp.where(valid, logits, -1e30)
    m = jnp.max(logits, axis=1, keepdims=True)
    p = jnp.exp(logits - m)
    p = jnp.where(valid, p, 0.0)
    probs = p / jnp.sum(p, axis=1, keepdims=True)
    a1 = jnp.max(probs, axis=1, keepdims=True)
    e1 = jnp.min(jnp.where(probs >= a1, lane, EL), axis=1, keepdims=True)
    probs2 = jnp.where(lane == e1, -1.0, probs)
    a2 = jnp.max(probs2, axis=1, keepdims=True)
    e2 = jnp.min(jnp.where(probs2 >= a2, lane, EL), axis=1, keepdims=True)
    tot = a1 + a2
    w_ref[...] = jnp.where(lane == 0, a1 / tot,
                           jnp.where(lane == 1, a2 / tot, 0.0))
    e_ref[...] = jnp.where(lane == 0, e1, jnp.where(lane == 1, e2, 0))
    oh = ((e1 == lane).astype(jnp.float32) + (e2 == lane).astype(jnp.float32))
    oh = jnp.where(valid, oh, 0.0)
    seg = seg_ref[...]
    hist_ref[...] = lax.dot_general(seg, oh, (((1,), (0,)), ((), ())),
                                    preferred_element_type=jnp.float32)


def _router(x, gw_pad, seg):
    return pl.pallas_call(
        _router_body,
        out_shape=(
            jax.ShapeDtypeStruct((T, EL), jnp.float32),
            jax.ShapeDtypeStruct((T, EL), jnp.int32),
            jax.ShapeDtypeStruct((NW, EL), jnp.float32),
        ),
    )(x, gw_pad, seg)


def _dispatch_body(ids_hbm, hist_hbm, x_hbm, pos_hbm, be_hbm, xs_hbm,
                   ids_v, hist_v, pos_v, base_ref, idx_e, idx_o,
                   rows_v, be_v, sem):
    cid = lax.axis_index("c")
    sid = lax.axis_index("s")
    wid = sid * 2 + cid
    base_slot = wid * CH
    pltpu.sync_copy(ids_hbm.at[pl.ds(base_slot, CH)], ids_v)
    pltpu.sync_copy(hist_hbm, hist_v)
    lanei = lax.iota(jnp.int32, 16)

    def _tot_step(w, acc):
        return acc + hist_v[pl.ds(w * 16, 16)]
    nvec = lax.fori_loop(0, NW, _tot_step, jnp.zeros((16,), jnp.int32))
    pb = (nvec + (BM - 1)) >> BMLOG
    csum = plsc.cumsum(pb)
    bstart = csum - pb
    segstart = bstart << BMLOG
    prefix = lax.fori_loop(0, wid, _tot_step, jnp.zeros((16,), jnp.int32))
    base_vec = segstart + prefix

    for i in range(CH // 16):
        v = ids_v[pl.ds(i * 16, 16)]
        base_ref[...] = base_vec
        bgat = plsc.load_gather(base_ref, [v])
        rank = jnp.zeros((16,), jnp.int32)
        hv = jnp.zeros((16,), jnp.int32)
        for e in range(E):
            m = v == e
            mi = jnp.where(m, 1, 0)
            rank = jnp.where(m, plsc.cumsum(mi), rank)
            hv = jnp.where(lanei == e, jnp.sum(mi), hv)
        pos_v[pl.ds(i * 16, 16)] = bgat + rank - 1
        base_vec = base_vec + hv
    pltpu.sync_copy(pos_v, pos_hbm.at[pl.ds(base_slot, CH)])

    for i in range(TPW // 16):
        j2 = (lax.iota(jnp.int32, 16) + i * 16) * 2
        idx_e[pl.ds(i * 16, 16)] = plsc.load_gather(pos_v, [j2])
        idx_o[pl.ds(i * 16, 16)] = plsc.load_gather(pos_v, [j2 + 1])
    pltpu.sync_copy(x_hbm.at[pl.ds(wid * TPW, TPW)], rows_v)
    pltpu.async_copy(rows_v, xs_hbm.at[idx_e], sem).wait()
    pltpu.async_copy(rows_v, xs_hbm.at[idx_o], sem).wait()

    @pl.when(wid == 0)
    def _():
        used = jnp.sum(jnp.where(lanei == E - 1, csum, 0))
        for i in range(BE_PAD // 16):
            bidx = lax.iota(jnp.int32, 16) + i * 16
            acc = jnp.zeros((16,), jnp.int32)
            for e in range(E):
                se = jnp.sum(jnp.where(lanei == e, bstart, 0))
                acc = acc + jnp.where(bidx >= se, 1, 0)
            bev = jnp.maximum(acc - 1, 0)
            be_v[pl.ds(i * 16, 16)] = jnp.where(bidx == NB, used, bev)
        pltpu.sync_copy(be_v, be_hbm)


def _dispatch(ids, hist, x):
    mesh = plsc.VectorSubcoreMesh(core_axis_name="c", subcore_axis_name="s")
    fn = pl.kernel(
        _dispatch_body,
        out_type=(
            jax.ShapeDtypeStruct((NSLOT,), jnp.int32),
            jax.ShapeDtypeStruct((BE_PAD,), jnp.int32),
            jax.ShapeDtypeStruct((S, H), jnp.float32),
        ),
        mesh=mesh,
        scratch_types=(
            pltpu.VMEM((CH,), jnp.int32),
            pltpu.VMEM((NW * 16,), jnp.int32),
            pltpu.VMEM((CH,), jnp.int32),
            pltpu.VMEM((16,), jnp.int32),
            pltpu.VMEM((TPW,), jnp.int32),
            pltpu.VMEM((TPW,), jnp.int32),
            pltpu.VMEM((TPW, H), jnp.float32),
            pltpu.VMEM((BE_PAD,), jnp.int32),
            pltpu.SemaphoreType.DMA,
        ),
        compiler_params=pltpu.CompilerParams(needs_layout_passes=False),
    )
    return fn(ids, hist, x)


def _ffn_body(be_ref, xs_ref, w1_ref, w3_ref, w2_ref, out_ref):
    i = pl.program_id(1)

    @pl.when(i < be_ref[NB])
    def _():
        x = xs_ref[...]
        w1b = w1_ref[0]
        w3b = w3_ref[0]
        w2b = w2_ref[0]
        t1 = lax.dot_general(x, w1b, (((1,), (1,)), ((), ())),
                             preferred_element_type=jnp.float32)
        t3 = lax.dot_general(x, w3b, (((1,), (1,)), ((), ())),
                             preferred_element_type=jnp.float32)
        h = (t1 / (1.0 + jnp.exp(-t1))) * t3
        out_ref[0] = lax.dot_general(
            h, w2b, (((1,), (1,)), ((), ())),
            preferred_element_type=jnp.float32)


def _ffn(be, xs, w1, w3, w2):
    grid_spec = pltpu.PrefetchScalarGridSpec(
        num_scalar_prefetch=1,
        grid=(2, NB),
        in_specs=[
            pl.BlockSpec((BM, H),
                         lambda j, i, be: (jnp.minimum(i, be[NB] - 1), 0)),
            pl.BlockSpec((1, F2, H),
                         lambda j, i, be: (be[jnp.minimum(i, be[NB] - 1)], j, 0)),
            pl.BlockSpec((1, F2, H),
                         lambda j, i, be: (be[jnp.minimum(i, be[NB] - 1)], j, 0)),
            pl.BlockSpec((1, H, F2),
                         lambda j, i, be: (be[jnp.minimum(i, be[NB] - 1)], 0, j)),
        ],
        out_specs=pl.BlockSpec(
            (1, BM, H), lambda j, i, be: (j, jnp.minimum(i, be[NB] - 1), 0)),
    )
    return pl.pallas_call(
        _ffn_body,
        grid_spec=grid_spec,
        out_shape=jax.ShapeDtypeStruct((2, S, H), jnp.float32),
        compiler_params=pltpu.CompilerParams(
            dimension_semantics=("arbitrary", "arbitrary"),
            vmem_limit_bytes=60 * 1024 * 1024),
    )(be, xs, w1, w3, w2)


def _combine_body(ys_hbm, pos_hbm, w_hbm, out_hbm,
                  pos_v, w_v, idx_v, idx2_v, rows_v, rows2_v, out_v, sem):
    cid = lax.axis_index("c")
    sid = lax.axis_index("s")
    wid = sid * 2 + cid
    base_slot = wid * CH
    base_tok = wid * TPW
    pltpu.sync_copy(pos_hbm.at[pl.ds(base_slot, CH)], pos_v)
    pltpu.sync_copy(w_hbm.at[pl.ds(base_slot, CH)], w_v)
    for g in range(4):
        for i in range(2):
            pv = pos_v[pl.ds(g * 32 + i * 16, 16)]
            idx_v[pl.ds(i * 16, 16)] = pv
            idx2_v[pl.ds(i * 16, 16)] = pv + S
        cp1 = pltpu.async_copy(ys_hbm.at[idx_v], rows_v, sem)
        cp2 = pltpu.async_copy(ys_hbm.at[idx2_v], rows2_v, sem)
        cp1.wait()
        cp2.wait()
        lanei = lax.iota(jnp.int32, 16)
        for r in range(16):
            s0 = g * 32 + 2 * r
            wchunk = w_v[pl.ds((s0 // 16) * 16, 16)]
            w0 = jnp.sum(jnp.where(lanei == s0 % 16, wchunk, 0.0))
            w1_ = jnp.sum(jnp.where(lanei == s0 % 16 + 1, wchunk, 0.0))

            def _col(c, _):
                for u in range(4):
                    o = c * 64 + u * 16
                    a = rows_v[2 * r, pl.ds(o, 16)] + rows2_v[2 * r, pl.ds(o, 16)]
                    b = rows_v[2 * r + 1, pl.ds(o, 16)] + rows2_v[2 * r + 1, pl.ds(o, 16)]
                    out_v[r, pl.ds(o, 16)] = w0 * a + w1_ * b
                return 0
            lax.fori_loop(0, H // 64, _col, 0)
        pltpu.sync_copy(out_v, out_hbm.at[pl.ds(base_tok + g * 16, 16)])


def _combine(ys, pos, w):
    mesh = plsc.VectorSubcoreMesh(core_axis_name="c", subcore_axis_name="s")
    fn = pl.kernel(
        _combine_body,
        out_type=jax.ShapeDtypeStruct((T, H), jnp.float32),
        mesh=mesh,
        scratch_types=(
            pltpu.VMEM((CH,), jnp.int32),
            pltpu.VMEM((CH,), jnp.float32),
            pltpu.VMEM((32,), jnp.int32),
            pltpu.VMEM((32,), jnp.int32),
            pltpu.VMEM((32, H), jnp.float32),
            pltpu.VMEM((32, H), jnp.float32),
            pltpu.VMEM((16, H), jnp.float32),
            pltpu.SemaphoreType.DMA,
        ),
        compiler_params=pltpu.CompilerParams(needs_layout_passes=False),
    )
    return fn(ys, pos, w)


@jax.jit
def _run(hidden_states, gate_w, w1, w2, w3):
    x = hidden_states.reshape(T, H)
    gw_pad = jnp.zeros((EL, H), jnp.float32).at[:E].set(gate_w)
    tok = lax.broadcasted_iota(jnp.int32, (NW, T), 1)
    chunk = lax.broadcasted_iota(jnp.int32, (NW, T), 0)
    seg = (tok // TPW == chunk).astype(jnp.float32)
    wout, eout, hist = _router(x, gw_pad, seg)
    topw = wout[:, :K].reshape(-1)
    ids = eout[:, :K].reshape(-1)
    hist_i = hist[:, :16].astype(jnp.int32).reshape(-1)
    pos, be, xs = _dispatch(ids, hist_i, x)
    ys = _ffn(be, xs, w1, w3, w2)
    final = _combine(ys.reshape(2 * S, H), pos, topw)
    return final.reshape(1, T, H)


def kernel(hidden_states, gate_w, w1, w2, w3):
    return _run(hidden_states, gate_w, w1, w2, w3)
